# baseline (speedup 1.0000x reference)
import jax
import jax.numpy as jnp
from jax.experimental import pallas as pl
from jax.experimental.pallas import tpu as pltpu

B = 1024
IMG = 28
FEAT = 64
OUT = 10
_F32 = jnp.float32
_BF = jnp.bfloat16
NCHUNK = 8
CHUNK = B // NCHUNK


def _fused(x1_ref, W_att_ref, b_att_ref, W_key_ref, W_query_ref, W_value_ref,
           W_skip_ref, b_conv_ref, gamma_ref, beta_ref, W_fc_ref, b_fc_ref,
           out_ref, x2_scr):
    i = pl.program_id(0)

    x1 = x1_ref[...].astype(_BF)
    x2_scr[pl.ds(i * CHUNK, CHUNK), :] = (
        jnp.dot(x1, W_att_ref[...].astype(_BF), preferred_element_type=_F32)
        + b_att_ref[...])

    @pl.when(i == NCHUNK - 1)
    def _tail():
        x2 = x2_scr[...]
        x2b = x2.astype(_BF)
        W_kq = (W_key_ref[...] + W_query_ref[...]).astype(_BF)
        kq = jnp.dot(x2b, W_kq, preferred_element_type=_F32)
        v = jnp.dot(x2b, W_value_ref[...].astype(_BF), preferred_element_type=_F32)
        skip = jnp.dot(x2b, W_skip_ref[...].astype(_BF), preferred_element_type=_F32)
        x4 = jax.nn.relu(skip + b_conv_ref[...]
                         + jnp.float32(B) * jax.nn.sigmoid(kq) * v)

        mean = jnp.mean(x4, axis=0, keepdims=True)
        var = jnp.mean((x4 - mean) ** 2, axis=0, keepdims=True)
        xn = ((x4 - mean) * jax.lax.rsqrt(var + 1e-5) * gamma_ref[...]
              + beta_ref[...])

        r = jax.lax.broadcasted_iota(jnp.int32, (FEAT, FEAT // 2), 0)
        c = jax.lax.broadcasted_iota(jnp.int32, (FEAT, FEAT // 2), 1)
        s_even = (r == 2 * c).astype(_F32)
        s_odd = (r == 2 * c + 1).astype(_F32)
        xp = jnp.maximum(jnp.dot(xn, s_even, preferred_element_type=_F32),
                         jnp.dot(xn, s_odd, preferred_element_type=_F32))

        xpb = xp.astype(_BF)
        logits = jax.lax.dot_general(xpb, xpb, (((1,), (1,)), ((), ())),
                                     preferred_element_type=_F32)
        att = jax.nn.sigmoid(logits)
        rowsum = jnp.sum(att, axis=1, keepdims=True)
        x5 = jnp.dot(att.astype(_BF), xpb, preferred_element_type=_F32) / rowsum

        out_ref[...] = (jnp.dot(x5 + xp, W_fc_ref[...],
                                preferred_element_type=_F32) + b_fc_ref[...])


def kernel(x, train, W_att, b_att, W_key, W_query, W_value, W_skip, b_conv,
           gamma, beta, W_fc, b_fc):
    del train
    Bs = x.shape[0]
    x1 = x.reshape(Bs, IMG * IMG)

    def _full(a):
        return pl.BlockSpec(a.shape, lambda i: tuple(0 for _ in a.shape))

    args = (x1, W_att, b_att.reshape(1, FEAT), W_key, W_query, W_value, W_skip,
            b_conv.reshape(1, FEAT), gamma.reshape(1, FEAT),
            beta.reshape(1, FEAT), W_fc, b_fc.reshape(1, OUT))
    in_specs = [pl.BlockSpec((CHUNK, IMG * IMG), lambda i: (i, 0))]
    in_specs += [_full(a) for a in args[1:]]
    return pl.pallas_call(
        _fused,
        grid=(NCHUNK,),
        in_specs=in_specs,
        out_specs=pl.BlockSpec((Bs, OUT), lambda i: (0, 0)),
        out_shape=jax.ShapeDtypeStruct((Bs, OUT), _F32),
        scratch_shapes=[pltpu.VMEM((B, FEAT), _F32)],
    )(*args)
